# baseline (speedup 1.0000x reference)
import dataclasses
import functools

import jax
import jax.numpy as jnp
from jax import lax
from jax.experimental import pallas as pl
from jax.experimental.pallas import tpu as pltpu
from jax.experimental.pallas import tpu_sc as plsc

ALPHA = 0.9
LANES = 16
NC = 2
NS = 16
CH = 128


def _deg_body(np_pad, chunks, col_hbm, ew_hbm, out_hbm,
              colv0, ewv0, src0, colv1, ewv1, src1,
              colv2, ewv2, src2, colv3, ewv3, src3,
              zbuf, dacc, isem0, isem1, isem2, isem3, ssem0, ssem1):
    sets = ((colv0, ewv0, src0), (colv1, ewv1, src1),
            (colv2, ewv2, src2), (colv3, ewv3, src3))
    isems = (isem0, isem1, isem2, isem3)
    ssems = (ssem0, ssem1)
    c = lax.axis_index("c")
    s = lax.axis_index("s")
    nps = np_pad // NS
    zr = zbuf.shape[0]

    @pl.loop(0, zr)
    def _(i):
        zbuf[i, pl.ds(0, LANES)] = jnp.zeros((LANES,), jnp.float32)

    @pl.loop(0, nps // zr)
    def _(k):
        pltpu.sync_copy(zbuf, dacc.at[pl.ds(s * nps + k * zr, zr)])

    plsc.subcore_barrier()

    w = c * NS + s
    cbase = w * chunks
    npairs = chunks // 2

    def idx_load(ch, st, isem):
        b = (cbase + ch) * CH
        pltpu.async_copy(col_hbm.at[pl.ds(b, CH)], st[0], isem)
        pltpu.async_copy(ew_hbm.at[pl.ds(b, CH)], st[1], isem)

    def idx_wait(ch, st, isem):
        b = (cbase + ch) * CH
        pltpu.make_async_copy(col_hbm.at[pl.ds(b, CH)], st[0], isem).wait()
        pltpu.make_async_copy(ew_hbm.at[pl.ds(b, CH)], st[1], isem).wait()

    def build(st):
        @pl.loop(0, CH // LANES)
        def _(g):
            wv16 = st[1][pl.ds(g * LANES, LANES)]
            for l in range(LANES):
                st[2][g * LANES + l, pl.ds(0, LANES)] = jnp.full(
                    (LANES,), wv16[l], jnp.float32)

    def one(jp, cur, nxt, cis, nis):
        for h in (0, 1):
            idx_wait(2 * jp + h, cur[h], cis[h])

        @pl.when(jp + 1 < npairs)
        def _():
            for h in (0, 1):
                idx_load(2 * (jp + 1) + h, nxt[h], nis[h])
        descs = []
        for h in (0, 1):
            build(cur[h])
            descs.append(pltpu.async_copy(
                cur[h][2], dacc.at[cur[h][0]], ssems[h], add=True))
        for d in descs:
            d.wait()

    pA, pB = (sets[0], sets[1]), (sets[2], sets[3])
    isA, isB = (isems[0], isems[1]), (isems[2], isems[3])

    pair0 = 0
    idx_load(0, pA[0], isA[0])
    idx_load(1, pA[1], isA[1])

    @pl.loop(0, npairs, step=2)
    def _(jp):
        one(jp, pA, pB, isA, isB)
        one(jp + 1, pB, pA, isB, isA)

    plsc.subcore_barrier()
    pltpu.sync_copy(dacc.at[pl.ds(s * nps, nps)],
                    out_hbm.at[c].at[pl.ds(s * nps, nps)])


def _agg_body(n, np_pad, chunks, hp_hbm, row_hbm, col_hbm, ew_hbm,
              out_hbm, *scr):
    sets = [tuple(scr[i * 5:i * 5 + 5]) for i in range(4)]
    zbuf, acc = scr[20], scr[21]
    isems, gsems = scr[22:26], scr[26:30]
    ssems = scr[30:32]
    c = lax.axis_index("c")
    s = lax.axis_index("s")
    nps = np_pad // NS
    zr = zbuf.shape[0]

    @pl.loop(0, zr)
    def _(i):
        zbuf[i, pl.ds(0, LANES)] = jnp.zeros((LANES,), jnp.float32)
        zbuf[i, pl.ds(LANES, LANES)] = jnp.zeros((LANES,), jnp.float32)

    @pl.loop(0, nps // zr)
    def _(k):
        pltpu.sync_copy(zbuf, acc.at[pl.ds(s * nps + k * zr, zr)])

    plsc.subcore_barrier()

    coff = c * n
    cbase = s * chunks
    npairs = chunks // 2

    def idx_load(ch, st, isem):
        b = (cbase + ch) * CH
        pltpu.async_copy(row_hbm.at[pl.ds(b, CH)], st[0], isem)
        pltpu.async_copy(col_hbm.at[pl.ds(b, CH)], st[1], isem)
        pltpu.async_copy(ew_hbm.at[pl.ds(b, CH)], st[2], isem)

    def idx_wait(ch, st, isem):
        b = (cbase + ch) * CH
        pltpu.make_async_copy(row_hbm.at[pl.ds(b, CH)], st[0], isem).wait()
        pltpu.make_async_copy(col_hbm.at[pl.ds(b, CH)], st[1], isem).wait()
        pltpu.make_async_copy(ew_hbm.at[pl.ds(b, CH)], st[2], isem).wait()

    def adjust(st):
        @pl.loop(0, CH // LANES)
        def _(t):
            st[0][pl.ds(t * LANES, LANES)] = (
                st[0][pl.ds(t * LANES, LANES)] + coff)

    def scale(st):
        @pl.loop(0, CH // LANES)
        def _(g):
            wv16 = st[2][pl.ds(g * LANES, LANES)]
            for l in range(LANES):
                wv = jnp.full((LANES,), wv16[l], jnp.float32)
                e = g * LANES + l
                v = st[3][e, pl.ds(0, LANES)]
                lo = plsc.bitcast(jnp.left_shift(v, 16), jnp.float32)
                hi = plsc.bitcast(
                    jnp.bitwise_and(v, jnp.int32(-65536)), jnp.float32)
                st[4][e, pl.ds(0, LANES)] = lo * wv
                st[4][e, pl.ds(LANES, LANES)] = hi * wv

    def pair_idx_load(jp, pr, sems):
        for h in (0, 1):
            idx_load(2 * jp + h, pr[h], sems[h])

    def pair_prep_gather(jp, pr, isms, gsms):
        for h in (0, 1):
            idx_wait(2 * jp + h, pr[h], isms[h])
            adjust(pr[h])
            pltpu.async_copy(hp_hbm.at[pr[h][0]], pr[h][3], gsms[h])

    def one(jp, cur, nxt, csem, nsem):
        cis, cgs = csem
        nis, ngs = nsem

        @pl.when(jp + 1 < npairs)
        def _():
            pair_prep_gather(jp + 1, nxt, nis, ngs)
        descs = []
        for h in (0, 1):
            pltpu.make_async_copy(
                hp_hbm.at[cur[h][0]], cur[h][3], cgs[h]).wait()
            scale(cur[h])
            descs.append(pltpu.async_copy(
                cur[h][4], acc.at[cur[h][1]], ssems[h], add=True))
        for d in descs:
            d.wait()

        @pl.when(jp + 2 < npairs)
        def _():
            pair_idx_load(jp + 2, cur, cis)

    pA, pB = (sets[0], sets[1]), (sets[2], sets[3])
    semA = ((isems[0], isems[1]), (gsems[0], gsems[1]))
    semB = ((isems[2], isems[3]), (gsems[2], gsems[3]))

    pair_idx_load(0, pA, semA[0])
    pair_prep_gather(0, pA, semA[0], semA[1])
    pair_idx_load(1, pB, semB[0])

    @pl.loop(0, npairs, step=2)
    def _(jp):
        one(jp, pA, pB, semA, semB)
        one(jp + 1, pB, pA, semB, semA)

    plsc.subcore_barrier()
    pltpu.sync_copy(acc.at[pl.ds(s * nps, nps)],
                    out_hbm.at[c].at[pl.ds(s * nps, nps)])


def _make_sc_kernels(n, np_pad, ep):
    mesh = plsc.VectorSubcoreMesh(core_axis_name="c", subcore_axis_name="s")
    cp = pltpu.CompilerParams(use_tc_tiling_on_sc=False)
    zr = 136
    deg_chunks = ep // (CH * NC * NS)
    agg_chunks = ep // (CH * NS)
    half = 32

    deg_scr = []
    for _ in range(4):
        deg_scr += [pltpu.VMEM((CH,), jnp.int32),
                    pltpu.VMEM((CH,), jnp.float32),
                    pltpu.VMEM((CH, LANES), jnp.float32)]
    deg_scr += [pltpu.VMEM((zr, LANES), jnp.float32),
                pltpu.VMEM_SHARED((np_pad, LANES), jnp.float32)]
    deg_scr += [pltpu.SemaphoreType.DMA] * 6
    deg_k = pl.kernel(
        functools.partial(_deg_body, np_pad, deg_chunks),
        out_type=jax.ShapeDtypeStruct((NC, np_pad, LANES), jnp.float32),
        mesh=mesh,
        scratch_types=deg_scr,
        compiler_params=cp,
    )

    agg_scr = []
    for _ in range(4):
        agg_scr += [pltpu.VMEM((CH,), jnp.int32),
                    pltpu.VMEM((CH,), jnp.int32),
                    pltpu.VMEM((CH,), jnp.float32),
                    pltpu.VMEM((CH, LANES), jnp.int32),
                    pltpu.VMEM((CH, half), jnp.float32)]
    agg_scr += [pltpu.VMEM((zr, half), jnp.float32),
                pltpu.VMEM_SHARED((np_pad, half), jnp.float32)]
    agg_scr += [pltpu.SemaphoreType.DMA] * 10
    agg_cp = cp
    if "needs_layout_passes" in pltpu.CompilerParams.__dataclass_fields__:
        agg_cp = dataclasses.replace(cp, needs_layout_passes=False)
    agg_k = pl.kernel(
        functools.partial(_agg_body, n, np_pad, agg_chunks),
        out_type=jax.ShapeDtypeStruct((NC, np_pad, half), jnp.float32),
        mesh=mesh,
        scratch_types=agg_scr,
        compiler_params=agg_cp,
    )
    return deg_k, agg_k


def _h0_body(x_ref, w0_ref, b0_ref, h0_ref):
    h = jnp.dot(x_ref[...], w0_ref[...], preferred_element_type=jnp.float32)
    h0_ref[...] = jnp.maximum(h + b0_ref[...], 0.0)


def _dis(d_ref):
    deg = d_ref[0, :, 0] + d_ref[1, :, 0]
    safe = jnp.where(deg > 0, deg, 1.0)
    return jnp.where(deg > 0, 1.0 / jnp.sqrt(safe), 0.0)[:, None]


def _pack_half(m):
    return jnp.stack([m[:, :16], m[:, 16:]], axis=-1).reshape(
        m.shape[0], 32).astype(jnp.bfloat16)


def _scale_body(h_ref, d_ref, hp_ref):
    dis = _dis(d_ref)
    h = h_ref[...]
    hp_ref[0] = _pack_half(dis * h[:, :32])
    hp_ref[1] = _pack_half(dis * h[:, 32:])


def _layer_body(a_ref, d_ref, x0_ref, pr_ref, w_ref, y_ref, hp_ref):
    dis = _dis(d_ref)
    agg = jnp.concatenate([a_ref[0], a_ref[1]], axis=1) * dis
    hh = (1.0 - ALPHA) * agg + ALPHA * x0_ref[...]
    mm = jnp.dot(hh, w_ref[...], preferred_element_type=jnp.float32)
    y = pr_ref[...] + jnp.maximum(mm, 0.0)
    y_ref[...] = y
    hp_ref[0] = _pack_half(dis * y[:, :32])
    hp_ref[1] = _pack_half(dis * y[:, 32:])


def _make_tc_kernels(n, d_in, d_f):
    bn = 2000
    grid = (n // bn,)
    half = 32

    def rb(shape, imap):
        return pl.BlockSpec(shape, imap)

    h0_call = pl.pallas_call(
        _h0_body,
        grid=grid,
        in_specs=[rb((bn, d_in), lambda i: (i, 0)),
                  rb((d_in, d_f), lambda i: (0, 0)),
                  rb((1, d_f), lambda i: (0, 0))],
        out_specs=rb((bn, d_f), lambda i: (i, 0)),
        out_shape=jax.ShapeDtypeStruct((n, d_f), jnp.float32),
    )

    scale_call = pl.pallas_call(
        _scale_body,
        grid=grid,
        in_specs=[rb((bn, d_f), lambda i: (i, 0)),
                  rb((NC, bn, LANES), lambda i: (0, i, 0))],
        out_specs=rb((NC, bn, half), lambda i: (0, i, 0)),
        out_shape=jax.ShapeDtypeStruct((NC, n, half), jnp.bfloat16),
    )

    layer_call = pl.pallas_call(
        _layer_body,
        grid=grid,
        in_specs=[rb((NC, bn, half), lambda i: (0, i, 0)),
                  rb((NC, bn, LANES), lambda i: (0, i, 0)),
                  rb((bn, d_f), lambda i: (i, 0)),
                  rb((bn, d_f), lambda i: (i, 0)),
                  rb((d_f, d_f), lambda i: (0, 0))],
        out_specs=[rb((bn, d_f), lambda i: (i, 0)),
                   rb((NC, bn, half), lambda i: (0, i, 0))],
        out_shape=[jax.ShapeDtypeStruct((n, d_f), jnp.float32),
                   jax.ShapeDtypeStruct((NC, n, half), jnp.bfloat16)],
    )
    return h0_call, scale_call, layer_call


def kernel(x, edge_index, edge_weight, edge_attr, W0, b0, W1, W2):
    del edge_attr
    n, d_in = x.shape
    d_f = W0.shape[1]
    e = edge_weight.shape[0]

    gran = CH * NC * NS
    ep = ((e + gran - 1) // gran) * gran
    pad = ep - e
    row = jnp.concatenate([edge_index[0], jnp.zeros((pad,), jnp.int32)])
    col = jnp.concatenate([edge_index[1], jnp.zeros((pad,), jnp.int32)])
    ew = jnp.concatenate([edge_weight, jnp.zeros((pad,), jnp.float32)])

    np_pad = ((n + 8 * NS - 1) // (8 * NS)) * (8 * NS)
    deg_k, agg_k = _make_sc_kernels(n, np_pad, ep)
    h0_call, scale_call, layer_call = _make_tc_kernels(n, d_in, d_f)

    dpart = deg_k(col, ew)
    h0 = h0_call(x, W0, b0.reshape(1, d_f))
    hp1 = scale_call(h0, dpart)
    hp1i = jax.lax.bitcast_convert_type(
        hp1.reshape(NC * n, LANES, 2), jnp.int32)
    acc1 = agg_k(hp1i, row, col, ew)
    out1, hp2 = layer_call(acc1, dpart, h0, h0, W1)
    hp2i = jax.lax.bitcast_convert_type(
        hp2.reshape(NC * n, LANES, 2), jnp.int32)
    acc2 = agg_k(hp2i, row, col, ew)
    out2, _ = layer_call(acc2, dpart, h0, out1, W2)
    return out2


# speedup vs baseline: 3.1069x; 3.1069x over previous
import functools

import jax
import jax.numpy as jnp
from jax import lax
from jax.experimental import pallas as pl
from jax.experimental.pallas import tpu as pltpu
from jax.experimental.pallas import tpu_sc as plsc

ALPHA = 0.9
LANES = 16
NC = 2
NS = 16
CH = 128


def _deg_body(np_pad, chunks, col_hbm, ew_hbm, out_hbm,
              colv0, ewv0, src0, colv1, ewv1, src1,
              colv2, ewv2, src2, colv3, ewv3, src3,
              zbuf, dacc, isem0, isem1, isem2, isem3, ssem0, ssem1):
    sets = ((colv0, ewv0, src0), (colv1, ewv1, src1),
            (colv2, ewv2, src2), (colv3, ewv3, src3))
    isems = (isem0, isem1, isem2, isem3)
    ssems = (ssem0, ssem1)
    c = lax.axis_index("c")
    s = lax.axis_index("s")
    nps = np_pad // NS
    zr = zbuf.shape[0]

    @pl.loop(0, zr)
    def _(i):
        zbuf[i, pl.ds(0, LANES)] = jnp.zeros((LANES,), jnp.float32)

    @pl.loop(0, nps // zr)
    def _(k):
        pltpu.sync_copy(zbuf, dacc.at[pl.ds(s * nps + k * zr, zr)])

    plsc.subcore_barrier()

    w = c * NS + s
    cbase = w * chunks
    npairs = chunks // 2

    def idx_load(ch, st, isem):
        b = (cbase + ch) * CH
        pltpu.async_copy(col_hbm.at[pl.ds(b, CH)], st[0], isem)
        pltpu.async_copy(ew_hbm.at[pl.ds(b, CH)], st[1], isem)

    def idx_wait(ch, st, isem):
        b = (cbase + ch) * CH
        pltpu.make_async_copy(col_hbm.at[pl.ds(b, CH)], st[0], isem).wait()
        pltpu.make_async_copy(ew_hbm.at[pl.ds(b, CH)], st[1], isem).wait()

    def build(st):
        @pl.loop(0, CH // LANES)
        def _(g):
            wv16 = st[1][pl.ds(g * LANES, LANES)]
            for l in range(LANES):
                st[2][g * LANES + l, pl.ds(0, LANES)] = jnp.full(
                    (LANES,), wv16[l], jnp.float32)

    def one(jp, cur, nxt, cis, nis):
        for h in (0, 1):
            idx_wait(2 * jp + h, cur[h], cis[h])

        @pl.when(jp + 1 < npairs)
        def _():
            for h in (0, 1):
                idx_load(2 * (jp + 1) + h, nxt[h], nis[h])
        descs = []
        for h in (0, 1):
            build(cur[h])
            descs.append(pltpu.async_copy(
                cur[h][2], dacc.at[cur[h][0]], ssems[h], add=True))
        for d in descs:
            d.wait()

    pA, pB = (sets[0], sets[1]), (sets[2], sets[3])
    isA, isB = (isems[0], isems[1]), (isems[2], isems[3])

    pair0 = 0
    idx_load(0, pA[0], isA[0])
    idx_load(1, pA[1], isA[1])

    @pl.loop(0, npairs, step=2)
    def _(jp):
        one(jp, pA, pB, isA, isB)
        one(jp + 1, pB, pA, isB, isA)

    plsc.subcore_barrier()
    pltpu.sync_copy(dacc.at[pl.ds(s * nps, nps)],
                    out_hbm.at[c].at[pl.ds(s * nps, nps)])


def _agg_body(n, np_pad, chunks, hp_hbm, row_hbm, col_hbm, ew_hbm,
              out_hbm, *scr):
    sets = [tuple(scr[i * 4:i * 4 + 4]) for i in range(4)]
    zbuf, acc = scr[16], scr[17]
    isems, gsems = scr[18:22], scr[22:26]
    ssems = scr[26:28]
    c = lax.axis_index("c")
    s = lax.axis_index("s")
    nps = np_pad // NS
    zr = zbuf.shape[0]

    @pl.loop(0, zr)
    def _(i):
        zbuf[i, pl.ds(0, LANES)] = jnp.zeros((LANES,), jnp.float32)
        zbuf[i, pl.ds(LANES, LANES)] = jnp.zeros((LANES,), jnp.float32)

    @pl.loop(0, nps // zr)
    def _(k):
        pltpu.sync_copy(zbuf, acc.at[pl.ds(s * nps + k * zr, zr)])

    plsc.subcore_barrier()

    coff = c * n
    cbase = s * chunks
    npairs = chunks // 2

    def idx_load(ch, st, isem):
        b = (cbase + ch) * CH
        pltpu.async_copy(row_hbm.at[pl.ds(b, CH)], st[0], isem)
        pltpu.async_copy(col_hbm.at[pl.ds(b, CH)], st[1], isem)
        pltpu.async_copy(ew_hbm.at[pl.ds(b, CH)], st[2], isem)

    def idx_wait(ch, st, isem):
        b = (cbase + ch) * CH
        pltpu.make_async_copy(row_hbm.at[pl.ds(b, CH)], st[0], isem).wait()
        pltpu.make_async_copy(col_hbm.at[pl.ds(b, CH)], st[1], isem).wait()
        pltpu.make_async_copy(ew_hbm.at[pl.ds(b, CH)], st[2], isem).wait()

    def adjust(st):
        @pl.loop(0, CH // LANES)
        def _(t):
            st[0][pl.ds(t * LANES, LANES)] = (
                st[0][pl.ds(t * LANES, LANES)] + coff)

    def scale(st):
        @pl.loop(0, CH // LANES)
        def _(g):
            wv16 = st[2][pl.ds(g * LANES, LANES)]
            for l in range(LANES):
                wv = jnp.full((LANES,), wv16[l], jnp.float32)
                e = g * LANES + l
                st[3][e, pl.ds(0, LANES)] = st[3][e, pl.ds(0, LANES)] * wv
                st[3][e, pl.ds(LANES, LANES)] = (
                    st[3][e, pl.ds(LANES, LANES)] * wv)

    def pair_idx_load(jp, pr, sems):
        for h in (0, 1):
            idx_load(2 * jp + h, pr[h], sems[h])

    def pair_prep_gather(jp, pr, isms, gsms):
        for h in (0, 1):
            idx_wait(2 * jp + h, pr[h], isms[h])
            adjust(pr[h])
            pltpu.async_copy(hp_hbm.at[pr[h][0]], pr[h][3], gsms[h])

    def one(jp, cur, nxt, csem, nsem):
        cis, cgs = csem
        nis, ngs = nsem

        @pl.when(jp + 1 < npairs)
        def _():
            pair_prep_gather(jp + 1, nxt, nis, ngs)
        descs = []
        for h in (0, 1):
            pltpu.make_async_copy(
                hp_hbm.at[cur[h][0]], cur[h][3], cgs[h]).wait()
            scale(cur[h])
            descs.append(pltpu.async_copy(
                cur[h][3], acc.at[cur[h][1]], ssems[h], add=True))
        for d in descs:
            d.wait()

        @pl.when(jp + 2 < npairs)
        def _():
            pair_idx_load(jp + 2, cur, cis)

    pA, pB = (sets[0], sets[1]), (sets[2], sets[3])
    semA = ((isems[0], isems[1]), (gsems[0], gsems[1]))
    semB = ((isems[2], isems[3]), (gsems[2], gsems[3]))

    pair_idx_load(0, pA, semA[0])
    pair_prep_gather(0, pA, semA[0], semA[1])
    pair_idx_load(1, pB, semB[0])

    @pl.loop(0, npairs, step=2)
    def _(jp):
        one(jp, pA, pB, semA, semB)
        one(jp + 1, pB, pA, semB, semA)

    plsc.subcore_barrier()
    pltpu.sync_copy(acc.at[pl.ds(s * nps, nps)],
                    out_hbm.at[c].at[pl.ds(s * nps, nps)])


def _make_sc_kernels(n, np_pad, ep):
    mesh = plsc.VectorSubcoreMesh(core_axis_name="c", subcore_axis_name="s")
    cp = pltpu.CompilerParams(use_tc_tiling_on_sc=False)
    zr = 136
    deg_chunks = ep // (CH * NC * NS)
    agg_chunks = ep // (CH * NS)
    half = 32

    deg_scr = []
    for _ in range(4):
        deg_scr += [pltpu.VMEM((CH,), jnp.int32),
                    pltpu.VMEM((CH,), jnp.float32),
                    pltpu.VMEM((CH, LANES), jnp.float32)]
    deg_scr += [pltpu.VMEM((zr, LANES), jnp.float32),
                pltpu.VMEM_SHARED((np_pad, LANES), jnp.float32)]
    deg_scr += [pltpu.SemaphoreType.DMA] * 6
    deg_k = pl.kernel(
        functools.partial(_deg_body, np_pad, deg_chunks),
        out_type=jax.ShapeDtypeStruct((NC, np_pad, LANES), jnp.float32),
        mesh=mesh,
        scratch_types=deg_scr,
        compiler_params=cp,
    )

    agg_scr = []
    for _ in range(4):
        agg_scr += [pltpu.VMEM((CH,), jnp.int32),
                    pltpu.VMEM((CH,), jnp.int32),
                    pltpu.VMEM((CH,), jnp.float32),
                    pltpu.VMEM((CH, half), jnp.float32)]
    agg_scr += [pltpu.VMEM((zr, half), jnp.float32),
                pltpu.VMEM_SHARED((np_pad, half), jnp.float32)]
    agg_scr += [pltpu.SemaphoreType.DMA] * 10
    agg_k = pl.kernel(
        functools.partial(_agg_body, n, np_pad, agg_chunks),
        out_type=jax.ShapeDtypeStruct((NC, np_pad, half), jnp.float32),
        mesh=mesh,
        scratch_types=agg_scr,
        compiler_params=cp,
    )
    return deg_k, agg_k


def _h0_body(x_ref, w0_ref, b0_ref, h0_ref):
    h = jnp.dot(x_ref[...], w0_ref[...], preferred_element_type=jnp.float32)
    h0_ref[...] = jnp.maximum(h + b0_ref[...], 0.0)


def _dis(d_ref):
    deg = d_ref[0, :, 0] + d_ref[1, :, 0]
    safe = jnp.where(deg > 0, deg, 1.0)
    return jnp.where(deg > 0, 1.0 / jnp.sqrt(safe), 0.0)[:, None]


def _scale_body(h_ref, d_ref, hp_ref):
    dis = _dis(d_ref)
    h = h_ref[...]
    hp_ref[0] = dis * h[:, :32]
    hp_ref[1] = dis * h[:, 32:]


def _layer_body(a_ref, d_ref, x0_ref, pr_ref, w_ref, y_ref, hp_ref):
    dis = _dis(d_ref)
    agg = jnp.concatenate([a_ref[0], a_ref[1]], axis=1) * dis
    hh = (1.0 - ALPHA) * agg + ALPHA * x0_ref[...]
    mm = jnp.dot(hh, w_ref[...], preferred_element_type=jnp.float32)
    y = pr_ref[...] + jnp.maximum(mm, 0.0)
    y_ref[...] = y
    hp_ref[0] = dis * y[:, :32]
    hp_ref[1] = dis * y[:, 32:]


def _make_tc_kernels(n, d_in, d_f):
    bn = 2000
    grid = (n // bn,)
    half = 32

    def rb(shape, imap):
        return pl.BlockSpec(shape, imap)

    h0_call = pl.pallas_call(
        _h0_body,
        grid=grid,
        in_specs=[rb((bn, d_in), lambda i: (i, 0)),
                  rb((d_in, d_f), lambda i: (0, 0)),
                  rb((1, d_f), lambda i: (0, 0))],
        out_specs=rb((bn, d_f), lambda i: (i, 0)),
        out_shape=jax.ShapeDtypeStruct((n, d_f), jnp.float32),
    )

    scale_call = pl.pallas_call(
        _scale_body,
        grid=grid,
        in_specs=[rb((bn, d_f), lambda i: (i, 0)),
                  rb((NC, bn, LANES), lambda i: (0, i, 0))],
        out_specs=rb((NC, bn, half), lambda i: (0, i, 0)),
        out_shape=jax.ShapeDtypeStruct((NC, n, half), jnp.float32),
    )

    layer_call = pl.pallas_call(
        _layer_body,
        grid=grid,
        in_specs=[rb((NC, bn, half), lambda i: (0, i, 0)),
                  rb((NC, bn, LANES), lambda i: (0, i, 0)),
                  rb((bn, d_f), lambda i: (i, 0)),
                  rb((bn, d_f), lambda i: (i, 0)),
                  rb((d_f, d_f), lambda i: (0, 0))],
        out_specs=[rb((bn, d_f), lambda i: (i, 0)),
                   rb((NC, bn, half), lambda i: (0, i, 0))],
        out_shape=[jax.ShapeDtypeStruct((n, d_f), jnp.float32),
                   jax.ShapeDtypeStruct((NC, n, half), jnp.float32)],
    )
    return h0_call, scale_call, layer_call


def kernel(x, edge_index, edge_weight, edge_attr, W0, b0, W1, W2):
    del edge_attr
    n, d_in = x.shape
    d_f = W0.shape[1]
    e = edge_weight.shape[0]

    gran = CH * NC * NS
    ep = ((e + gran - 1) // gran) * gran
    pad = ep - e
    row = jnp.concatenate([edge_index[0], jnp.zeros((pad,), jnp.int32)])
    col = jnp.concatenate([edge_index[1], jnp.zeros((pad,), jnp.int32)])
    ew = jnp.concatenate([edge_weight, jnp.zeros((pad,), jnp.float32)])

    np_pad = ((n + 8 * NS - 1) // (8 * NS)) * (8 * NS)
    deg_k, agg_k = _make_sc_kernels(n, np_pad, ep)
    h0_call, scale_call, layer_call = _make_tc_kernels(n, d_in, d_f)

    dpart = deg_k(col, ew)
    h0 = h0_call(x, W0, b0.reshape(1, d_f))
    hp1 = scale_call(h0, dpart)
    acc1 = agg_k(hp1.reshape(NC * n, 32), row, col, ew)
    out1, hp2 = layer_call(acc1, dpart, h0, h0, W1)
    acc2 = agg_k(hp2.reshape(NC * n, 32), row, col, ew)
    out2, _ = layer_call(acc2, dpart, h0, out1, W2)
    return out2
